# Initial kernel scaffold; baseline (speedup 1.0000x reference)
import jax
import jax.numpy as jnp
from jax.experimental import pallas as pl


def kernel(x, edge_index, W1, att_src1, att_dst1, b1, W2, att_src2, att_dst2, b2, fc_w, fc_b):
    raise NotImplementedError("write your pallas kernel here")


# speedup vs baseline: 1.0692x; 1.0692x over previous
import functools

import jax
import jax.numpy as jnp
from jax.experimental import pallas as pl
from jax.experimental.pallas import tpu as pltpu

_N = 10000
_E = 160000
_D = 256
_H = 4
_C = 256
_OUT = 128
_BN = 1000


def _build_aatt(att_src, att_dst):
    s = att_src.reshape(_H, _C)
    d = att_dst.reshape(_H, _C)
    eye = jnp.eye(_H, dtype=jnp.float32)
    a_s = (eye[:, None, :] * s[:, :, None]).reshape(_H * _C, _H)
    a_d = (eye[:, None, :] * d[:, :, None]).reshape(_H * _C, _H)
    return jnp.concatenate([a_s, a_d], axis=1)


def _proj_body(x_ref, w_ref, aatt_ref, h_ref, a_ref):
    h = jnp.dot(x_ref[...], w_ref[...], preferred_element_type=jnp.float32)
    h_ref[...] = h
    a_ref[...] = jnp.dot(h, aatt_ref[...], preferred_element_type=jnp.float32)


def _project(x, w, aatt):
    n, k = x.shape
    m = w.shape[1]
    return pl.pallas_call(
        _proj_body,
        grid=(n // _BN,),
        in_specs=[
            pl.BlockSpec((_BN, k), lambda i: (i, 0)),
            pl.BlockSpec((k, m), lambda i: (0, 0)),
            pl.BlockSpec((m, 2 * _H), lambda i: (0, 0)),
        ],
        out_specs=[
            pl.BlockSpec((_BN, m), lambda i: (i, 0)),
            pl.BlockSpec((_BN, 2 * _H), lambda i: (i, 0)),
        ],
        out_shape=[
            jax.ShapeDtypeStruct((n, m), jnp.float32),
            jax.ShapeDtypeStruct((n, 2 * _H), jnp.float32),
        ],
    )(x, w, aatt)


def _edge_phase(h, a, src, dst):
    a_src = a[:, :_H]
    a_dst = a[:, _H:]
    al = a_src[src] + a_dst[dst]
    al = jnp.where(al >= 0, al, 0.2 * al)
    w = jnp.exp(al)
    asum = jax.ops.segment_sum(w, dst, num_segments=_N)
    h3 = h.reshape(_N, _H, _C)
    msg = h3[src] * w[:, :, None]
    out = jax.ops.segment_sum(msg, dst, num_segments=_N)
    out = out / (asum + 1e-16)[:, :, None]
    return out.mean(axis=1)


def _elu(x):
    return jnp.where(x > 0, x, jnp.expm1(x))


def kernel(x, edge_index, W1, att_src1, att_dst1, b1, W2, att_src2, att_dst2,
           b2, fc_w, fc_b):
    loop = jnp.arange(_N, dtype=edge_index.dtype)
    src = jnp.concatenate([edge_index[0], loop])
    dst = jnp.concatenate([edge_index[1], loop])

    aatt1 = _build_aatt(att_src1, att_dst1)
    aatt2 = _build_aatt(att_src2, att_dst2)

    h1p, a1 = _project(x, W1, aatt1)
    h1 = _elu(_edge_phase(h1p, a1, src, dst) + b1)
    h2p, a2 = _project(h1, W2, aatt2)
    h2 = _elu(_edge_phase(h2p, a2, src, dst) + b2)
    g = h2.mean(axis=0, keepdims=True)
    return g @ fc_w + fc_b


# speedup vs baseline: 15.3410x; 14.3475x over previous
import functools

import jax
import jax.numpy as jnp
from jax import lax
from jax.experimental import pallas as pl
from jax.experimental.pallas import tpu as pltpu
from jax.experimental.pallas import tpu_sc as plsc

_N = 10000
_E = 160000
_D = 256
_H = 4
_C = 256
_OUT = 128
_BN = 1000

_EREAL = _E + _N
_EPAD = 172032
_BTA = _EPAD // 32
_NCA = _BTA // 128
_NPAD = 10240
_NPT = _NPAD // 16

_BTB = _EPAD // 16
_NSEC = 7
_SCH = 12
_NCB = _BTB // 128

_SC_PARAMS = pltpu.CompilerParams(needs_layout_passes=False,
                                  use_tc_tiling_on_sc=False)


def _edge_w_body(a_hbm, src_hbm, dst_hbm, w_hbm, asum_hbm,
                 a_tab, src_v, dst_v, stage, stage2, acc):
    c = lax.axis_index("c")
    s = lax.axis_index("s")
    wid = c * 16 + s
    ebase = wid * _BTA

    def _zero_stage(i, carry):
        stage[i, :] = jnp.zeros((16,), jnp.float32)
        return carry
    lax.fori_loop(0, 128, _zero_stage, 0)
    for k in range(5):
        pltpu.sync_copy(stage, acc.at[pl.ds(s * _NPT + k * 128, 128)])
    pltpu.sync_copy(a_hbm, a_tab)
    pltpu.sync_copy(src_hbm.at[wid], src_v)
    pltpu.sync_copy(dst_hbm.at[wid], dst_v)
    plsc.subcore_barrier()

    def _chunk(ci, carry):
        for i in range(8):
            src16 = src_v[ci, pl.ds(i * 16, 16)]
            dst16 = dst_v[ci, pl.ds(i * 16, 16)]
            ge = ebase + ci * 128 + i * 16 + lax.iota(jnp.int32, 16)
            valid = ge < _EREAL
            row = i * 16 + lax.iota(jnp.int32, 16)
            src8 = src16 * (2 * _H)
            dst8 = dst16 * (2 * _H)
            for h in range(_H):
                asrc = plsc.load_gather(a_tab, [src8 + h])
                adst = plsc.load_gather(a_tab, [dst8 + (_H + h)])
                al = asrc + adst
                al = jnp.maximum(al, 0.2 * al)
                w = jnp.where(valid, jnp.exp(al), 0.0)
                plsc.store_scatter(
                    stage, [row, jnp.full((16,), h, jnp.int32)], w)
                stage2[h, pl.ds(i * 16, 16)] = w
        pltpu.sync_copy(stage2, w_hbm.at[wid * _NCA + ci])
        pltpu.sync_copy(stage, acc.at[dst_v.at[ci]], add=True)
        return carry
    lax.fori_loop(0, _NCA, _chunk, 0)
    plsc.subcore_barrier()

    for k in range(5):
        pltpu.sync_copy(acc.at[pl.ds(s * _NPT + k * 128, 128)],
                        asum_hbm.at[c, pl.ds(s * _NPT + k * 128, 128)])


def _edge_weights(a, src_r, dst_r):
    mesh = plsc.VectorSubcoreMesh(core_axis_name="c", subcore_axis_name="s")
    f = pl.kernel(
        _edge_w_body,
        out_type=[jax.ShapeDtypeStruct((_EPAD // 128, 16, 128), jnp.float32),
                  jax.ShapeDtypeStruct((2, _NPAD, 16), jnp.float32)],
        mesh=mesh,
        scratch_types=[pltpu.VMEM((_N * 2 * _H,), jnp.float32),
                       pltpu.VMEM((_NCA, 128), jnp.int32),
                       pltpu.VMEM((_NCA, 128), jnp.int32),
                       pltpu.VMEM((128, 16), jnp.float32),
                       pltpu.VMEM((16, 128), jnp.float32),
                       pltpu.VMEM_SHARED((_NPAD, 16), jnp.float32)],
        compiler_params=_SC_PARAMS,
    )
    return f(a.reshape(-1), src_r, dst_r)


def _msg_body(h_hbm, src_hbm, dst_hbm, w_hbm, out_hbm,
              sidx, dstv, wv, gb0, acc, sem0):
    c = lax.axis_index("c")
    s = lax.axis_index("s")

    def _zero_gb(i, carry):
        for k in range(8):
            gb0[i, pl.ds(k * 16, 16)] = jnp.zeros((16,), jnp.float32)
        return carry
    lax.fori_loop(0, 128, _zero_gb, 0)

    for g_local in range(4):
        g = 4 * c + g_local
        head = 2 * c + (g_local // 2)
        for k in range(5):
            pltpu.sync_copy(gb0, acc.at[pl.ds(s * _NPT + k * 128, 128)])
        plsc.subcore_barrier()

        def _section(sec, carry):
            pltpu.sync_copy(src_hbm.at[s, sec], sidx)
            pltpu.sync_copy(dst_hbm.at[s, sec], dstv)
            pltpu.sync_copy(
                w_hbm.at[pl.ds(s * _NCB + sec * _SCH, _SCH), head], wv)

            def _tx(r, carry):
                for k in range(8):
                    v = sidx[r, pl.ds(k * 16, 16)]
                    sidx[r, pl.ds(k * 16, 16)] = v * 8 + g
                return carry
            lax.fori_loop(0, _SCH, _tx, 0)

            def _chunk(ci, carry):
                pltpu.sync_copy(h_hbm.at[sidx.at[ci]], gb0)

                def _mul(j16, carry2):
                    wvec = wv[ci, pl.ds(j16 * 16, 16)]
                    for jj in range(16):
                        r = j16 * 16 + jj
                        ws = wvec[jj]
                        for k in range(8):
                            gb0[r, pl.ds(k * 16, 16)] = (
                                gb0[r, pl.ds(k * 16, 16)] * ws)
                    return carry2
                lax.fori_loop(0, 8, _mul, 0)
                pltpu.sync_copy(gb0, acc.at[dstv.at[ci]], add=True)
                return carry
            lax.fori_loop(0, _SCH, _chunk, 0)
            return carry
        lax.fori_loop(0, _NSEC, _section, 0)
        plsc.subcore_barrier()
        pltpu.sync_copy(acc.at[pl.ds(s * _NPT, _NPT)],
                        out_hbm.at[g, pl.ds(s * _NPT, _NPT)])
        lax.fori_loop(0, 128, _zero_gb, 0)


def _message_pass(h_flat, src_rb, dst_rb, w2):
    mesh = plsc.VectorSubcoreMesh(core_axis_name="c", subcore_axis_name="s")
    f = pl.kernel(
        _msg_body,
        out_type=jax.ShapeDtypeStruct((8, _NPAD, 128), jnp.float32),
        mesh=mesh,
        scratch_types=[pltpu.VMEM((_SCH, 128), jnp.int32),
                       pltpu.VMEM((_SCH, 128), jnp.int32),
                       pltpu.VMEM((_SCH, 128), jnp.float32),
                       pltpu.VMEM((128, 128), jnp.float32),
                       pltpu.VMEM_SHARED((_NPAD, 128), jnp.float32),
                       pltpu.SemaphoreType.DMA],
        compiler_params=_SC_PARAMS,
    )
    return f(h_flat, src_rb, dst_rb, w2)


def _build_aatt(att_src, att_dst):
    sv = att_src.reshape(_H, _C)
    dv = att_dst.reshape(_H, _C)
    eye = jnp.eye(_H, dtype=jnp.float32)
    a_s = (eye[:, None, :] * sv[:, :, None]).reshape(_H * _C, _H)
    a_d = (eye[:, None, :] * dv[:, :, None]).reshape(_H * _C, _H)
    return jnp.concatenate([a_s, a_d], axis=1)


def _proj_body(x_ref, w_ref, aatt_ref, h_ref, a_ref):
    h = jnp.dot(x_ref[...], w_ref[...], preferred_element_type=jnp.float32)
    h_ref[...] = h
    a_ref[...] = jnp.dot(h, aatt_ref[...], preferred_element_type=jnp.float32)


def _project(x, w, aatt):
    n, k = x.shape
    m = w.shape[1]
    return pl.pallas_call(
        _proj_body,
        grid=(n // _BN,),
        in_specs=[
            pl.BlockSpec((_BN, k), lambda i: (i, 0)),
            pl.BlockSpec((k, m), lambda i: (0, 0)),
            pl.BlockSpec((m, 2 * _H), lambda i: (0, 0)),
        ],
        out_specs=[
            pl.BlockSpec((_BN, m), lambda i: (i, 0)),
            pl.BlockSpec((_BN, 2 * _H), lambda i: (i, 0)),
        ],
        out_shape=[
            jax.ShapeDtypeStruct((n, m), jnp.float32),
            jax.ShapeDtypeStruct((n, 2 * _H), jnp.float32),
        ],
    )(x, w, aatt)


def _normalize(raw_ref, asum_ref, b_ref):
    acc = None
    for h in range(_H):
        part = jnp.concatenate([raw_ref[2 * h], raw_ref[2 * h + 1]], axis=1)
        d = asum_ref[...][:, h:h + 1] + 1e-16
        t = part / d
        acc = t if acc is None else acc + t
    hcur = acc * (1.0 / _H) + b_ref[...]
    return jnp.where(hcur > 0, hcur, jnp.exp(jnp.minimum(hcur, 0.0)) - 1.0)


def _np_body(raw_ref, asum_ref, b_ref, w_ref, aatt_ref, h_ref, a_ref):
    hcur = _normalize(raw_ref, asum_ref, b_ref)
    h = jnp.dot(hcur, w_ref[...], preferred_element_type=jnp.float32)
    h_ref[...] = h
    a_ref[...] = jnp.dot(h, aatt_ref[...], preferred_element_type=jnp.float32)


def _norm_project(raw, asum, b, w, aatt):
    m = w.shape[1]
    return pl.pallas_call(
        _np_body,
        grid=(_N // _BN,),
        in_specs=[
            pl.BlockSpec((8, _BN, 128), lambda i: (0, i, 0)),
            pl.BlockSpec((_BN, _H), lambda i: (i, 0)),
            pl.BlockSpec((1, _C), lambda i: (0, 0)),
            pl.BlockSpec((_C, m), lambda i: (0, 0)),
            pl.BlockSpec((m, 2 * _H), lambda i: (0, 0)),
        ],
        out_specs=[
            pl.BlockSpec((_BN, m), lambda i: (i, 0)),
            pl.BlockSpec((_BN, 2 * _H), lambda i: (i, 0)),
        ],
        out_shape=[
            jax.ShapeDtypeStruct((_N, m), jnp.float32),
            jax.ShapeDtypeStruct((_N, 2 * _H), jnp.float32),
        ],
    )(raw, asum, b, w, aatt)


def _final_body(raw_ref, asum_ref, b_ref, fcw_ref, fcb_ref, out_ref, acc_ref):
    i = pl.program_id(0)
    hcur = _normalize(raw_ref, asum_ref, b_ref)
    ssum = jnp.sum(hcur, axis=0, keepdims=True)

    @pl.when(i == 0)
    def _():
        acc_ref[...] = ssum

    @pl.when(i > 0)
    def _():
        acc_ref[...] = acc_ref[...] + ssum

    @pl.when(i == (_N // _BN) - 1)
    def _():
        g = acc_ref[...] * (1.0 / _N)
        out_ref[...] = (jnp.dot(g, fcw_ref[...],
                                preferred_element_type=jnp.float32)
                        + fcb_ref[...])


def _final(raw, asum, b, fc_w, fc_b):
    return pl.pallas_call(
        _final_body,
        grid=(_N // _BN,),
        in_specs=[
            pl.BlockSpec((8, _BN, 128), lambda i: (0, i, 0)),
            pl.BlockSpec((_BN, _H), lambda i: (i, 0)),
            pl.BlockSpec((1, _C), lambda i: (0, 0)),
            pl.BlockSpec((_C, _OUT), lambda i: (0, 0)),
            pl.BlockSpec((1, _OUT), lambda i: (0, 0)),
        ],
        out_specs=pl.BlockSpec((1, _OUT), lambda i: (0, 0)),
        out_shape=jax.ShapeDtypeStruct((1, _OUT), jnp.float32),
        scratch_shapes=[pltpu.VMEM((1, _C), jnp.float32)],
    )(raw, asum, b, fc_w, fc_b)


def kernel(x, edge_index, W1, att_src1, att_dst1, b1, W2, att_src2, att_dst2,
           b2, fc_w, fc_b):
    loop = jnp.arange(_N, dtype=edge_index.dtype)
    src = jnp.concatenate([edge_index[0], loop])
    dst = jnp.concatenate([edge_index[1], loop])
    src_p = jnp.pad(src.astype(jnp.int32), (0, _EPAD - _EREAL))
    dst_p = jnp.pad(dst.astype(jnp.int32), (0, _EPAD - _EREAL))
    src_ra = src_p.reshape(32, _NCA, 128)
    dst_ra = dst_p.reshape(32, _NCA, 128)
    src_rb = src_p.reshape(16, _NSEC, _SCH, 128)
    dst_rb = dst_p.reshape(16, _NSEC, _SCH, 128)

    aatt1 = _build_aatt(att_src1, att_dst1)
    aatt2 = _build_aatt(att_src2, att_dst2)

    def layer(h_pre, a):
        w2, asum_p = _edge_weights(a, src_ra, dst_ra)
        asum = asum_p[0, :_N, :_H] + asum_p[1, :_N, :_H]
        raw = _message_pass(h_pre.reshape(_N * 2 * _H, 128),
                            src_rb, dst_rb, w2)
        return raw[:, :_N, :], asum

    h1p, a1 = _project(x, W1, aatt1)
    raw1, asum1 = layer(h1p, a1)
    h2p, a2 = _norm_project(raw1, asum1, b1.reshape(1, _C), W2, aatt2)
    raw2, asum2 = layer(h2p, a2)
    return _final(raw2, asum2, b2.reshape(1, _C), fc_w, fc_b.reshape(1, _OUT))


# speedup vs baseline: 19.8296x; 1.2926x over previous
import functools

import jax
import jax.numpy as jnp
from jax import lax
from jax.experimental import pallas as pl
from jax.experimental.pallas import tpu as pltpu
from jax.experimental.pallas import tpu_sc as plsc

_N = 10000
_E = 160000
_D = 256
_H = 4
_C = 256
_OUT = 128
_BN = 1000

_EREAL = _E + _N
_EPAD = 172032
_BTA = _EPAD // 32
_NCA = _BTA // 128
_NPAD = 10240
_NPT = _NPAD // 16

_BTB = _EPAD // 16
_NSEC = 7
_SCH = 12
_NCB = _BTB // 128

_SC_PARAMS = pltpu.CompilerParams(needs_layout_passes=False,
                                  use_tc_tiling_on_sc=False)


def _edge_w_body(a_hbm, src_hbm, dst_hbm, w_hbm, asum_hbm,
                 a_tab, src_v, dst_v, stage, stage2, acc):
    c = lax.axis_index("c")
    s = lax.axis_index("s")
    wid = c * 16 + s
    ebase = wid * _BTA

    def _zero_stage(i, carry):
        stage[i, :] = jnp.zeros((16,), jnp.float32)
        return carry
    lax.fori_loop(0, 128, _zero_stage, 0)
    for k in range(5):
        pltpu.sync_copy(stage, acc.at[pl.ds(s * _NPT + k * 128, 128)])
    pltpu.sync_copy(a_hbm, a_tab)
    pltpu.sync_copy(src_hbm.at[wid], src_v)
    pltpu.sync_copy(dst_hbm.at[wid], dst_v)
    plsc.subcore_barrier()

    def _chunk(ci, carry):
        for i in range(8):
            src16 = src_v[ci, pl.ds(i * 16, 16)]
            dst16 = dst_v[ci, pl.ds(i * 16, 16)]
            ge = ebase + ci * 128 + i * 16 + lax.iota(jnp.int32, 16)
            valid = ge < _EREAL
            row = i * 16 + lax.iota(jnp.int32, 16)
            src8 = src16 * (2 * _H)
            dst8 = dst16 * (2 * _H)
            for h in range(_H):
                asrc = plsc.load_gather(a_tab, [src8 + h])
                adst = plsc.load_gather(a_tab, [dst8 + (_H + h)])
                al = asrc + adst
                al = jnp.maximum(al, 0.2 * al)
                w = jnp.where(valid, jnp.exp(al), 0.0)
                plsc.store_scatter(
                    stage, [row, jnp.full((16,), h, jnp.int32)], w)
                stage2[h, pl.ds(i * 16, 16)] = w
        pltpu.sync_copy(stage2, w_hbm.at[wid * _NCA + ci])
        pltpu.sync_copy(stage, acc.at[dst_v.at[ci]], add=True)
        return carry
    lax.fori_loop(0, _NCA, _chunk, 0)
    plsc.subcore_barrier()

    for k in range(5):
        pltpu.sync_copy(acc.at[pl.ds(s * _NPT + k * 128, 128)],
                        asum_hbm.at[c, pl.ds(s * _NPT + k * 128, 128)])


def _edge_weights(a, src_r, dst_r):
    mesh = plsc.VectorSubcoreMesh(core_axis_name="c", subcore_axis_name="s")
    f = pl.kernel(
        _edge_w_body,
        out_type=[jax.ShapeDtypeStruct((_EPAD // 128, 16, 128), jnp.float32),
                  jax.ShapeDtypeStruct((2, _NPAD, 16), jnp.float32)],
        mesh=mesh,
        scratch_types=[pltpu.VMEM((_N * 2 * _H,), jnp.float32),
                       pltpu.VMEM((_NCA, 128), jnp.int32),
                       pltpu.VMEM((_NCA, 128), jnp.int32),
                       pltpu.VMEM((128, 16), jnp.float32),
                       pltpu.VMEM((16, 128), jnp.float32),
                       pltpu.VMEM_SHARED((_NPAD, 16), jnp.float32)],
        compiler_params=_SC_PARAMS,
    )
    return f(a.reshape(-1), src_r, dst_r)


def _msg_body(h_hbm, src_hbm, dst_hbm, w_hbm, out_hbm,
              sidx, dstv, wv, gb0, gb1, acc, gsem0, gsem1, ssem0, ssem1):
    c = lax.axis_index("c")
    s = lax.axis_index("s")

    def _zero_gb(i, carry):
        for k in range(8):
            gb0[i, pl.ds(k * 16, 16)] = jnp.zeros((16,), jnp.float32)
        return carry
    lax.fori_loop(0, 128, _zero_gb, 0)

    for g_local in range(4):
        g = 4 * c + g_local
        head = 2 * c + (g_local // 2)
        for k in range(5):
            pltpu.sync_copy(gb0, acc.at[pl.ds(s * _NPT + k * 128, 128)])
        plsc.subcore_barrier()

        def _section(sec, carry):
            pltpu.sync_copy(src_hbm.at[s, sec], sidx)
            pltpu.sync_copy(dst_hbm.at[s, sec], dstv)
            pltpu.sync_copy(
                w_hbm.at[pl.ds(s * _NCB + sec * _SCH, _SCH), head], wv)

            def _tx(r, carry):
                for k in range(8):
                    v = sidx[r, pl.ds(k * 16, 16)]
                    sidx[r, pl.ds(k * 16, 16)] = v * 8 + g
                return carry
            lax.fori_loop(0, _SCH, _tx, 0)

            def _mul(gb, ci):
                def _mul16(j16, carry2):
                    wvec = wv[ci, pl.ds(j16 * 16, 16)]
                    for jj in range(16):
                        r = j16 * 16 + jj
                        ws = wvec[jj]
                        for k in range(8):
                            gb[r, pl.ds(k * 16, 16)] = (
                                gb[r, pl.ds(k * 16, 16)] * ws)
                    return carry2
                lax.fori_loop(0, 8, _mul16, 0)

            pltpu.async_copy(h_hbm.at[sidx.at[0]], gb0, gsem0)

            def _pair(p, carry):
                ci0 = 2 * p
                ci1 = 2 * p + 1
                pltpu.make_async_copy(
                    h_hbm.at[sidx.at[ci0]], gb0, gsem0).wait()

                @pl.when(p > 0)
                def _():
                    pltpu.make_async_copy(
                        gb1, acc.at[dstv.at[ci0 - 1]], ssem1).wait()
                pltpu.async_copy(h_hbm.at[sidx.at[ci1]], gb1, gsem1)
                _mul(gb0, ci0)
                pltpu.async_copy(gb0, acc.at[dstv.at[ci0]], ssem0, add=True)
                pltpu.make_async_copy(
                    h_hbm.at[sidx.at[ci1]], gb1, gsem1).wait()
                pltpu.make_async_copy(
                    gb0, acc.at[dstv.at[ci0]], ssem0).wait()

                @pl.when(p < _SCH // 2 - 1)
                def _():
                    pltpu.async_copy(h_hbm.at[sidx.at[ci0 + 2]], gb0, gsem0)
                _mul(gb1, ci1)
                pltpu.async_copy(gb1, acc.at[dstv.at[ci1]], ssem1, add=True)
                return carry
            lax.fori_loop(0, _SCH // 2, _pair, 0)
            pltpu.make_async_copy(
                gb1, acc.at[dstv.at[_SCH - 1]], ssem1).wait()
            return carry
        lax.fori_loop(0, _NSEC, _section, 0)
        plsc.subcore_barrier()
        pltpu.sync_copy(acc.at[pl.ds(s * _NPT, _NPT)],
                        out_hbm.at[g, pl.ds(s * _NPT, _NPT)])
        lax.fori_loop(0, 128, _zero_gb, 0)


def _message_pass(h_flat, src_rb, dst_rb, w2):
    mesh = plsc.VectorSubcoreMesh(core_axis_name="c", subcore_axis_name="s")
    f = pl.kernel(
        _msg_body,
        out_type=jax.ShapeDtypeStruct((8, _NPAD, 128), jnp.float32),
        mesh=mesh,
        scratch_types=[pltpu.VMEM((_SCH, 128), jnp.int32),
                       pltpu.VMEM((_SCH, 128), jnp.int32),
                       pltpu.VMEM((_SCH, 128), jnp.float32),
                       pltpu.VMEM((128, 128), jnp.float32),
                       pltpu.VMEM((128, 128), jnp.float32),
                       pltpu.VMEM_SHARED((_NPAD, 128), jnp.float32),
                       pltpu.SemaphoreType.DMA,
                       pltpu.SemaphoreType.DMA,
                       pltpu.SemaphoreType.DMA,
                       pltpu.SemaphoreType.DMA],
        compiler_params=_SC_PARAMS,
    )
    return f(h_flat, src_rb, dst_rb, w2)


def _build_aatt(att_src, att_dst):
    sv = att_src.reshape(_H, _C)
    dv = att_dst.reshape(_H, _C)
    eye = jnp.eye(_H, dtype=jnp.float32)
    a_s = (eye[:, None, :] * sv[:, :, None]).reshape(_H * _C, _H)
    a_d = (eye[:, None, :] * dv[:, :, None]).reshape(_H * _C, _H)
    return jnp.concatenate([a_s, a_d], axis=1)


def _proj_body(x_ref, w_ref, aatt_ref, h_ref, a_ref):
    h = jnp.dot(x_ref[...], w_ref[...], preferred_element_type=jnp.float32)
    h_ref[...] = h
    a_ref[...] = jnp.dot(h, aatt_ref[...], preferred_element_type=jnp.float32)


def _project(x, w, aatt):
    n, k = x.shape
    m = w.shape[1]
    return pl.pallas_call(
        _proj_body,
        grid=(n // _BN,),
        in_specs=[
            pl.BlockSpec((_BN, k), lambda i: (i, 0)),
            pl.BlockSpec((k, m), lambda i: (0, 0)),
            pl.BlockSpec((m, 2 * _H), lambda i: (0, 0)),
        ],
        out_specs=[
            pl.BlockSpec((_BN, m), lambda i: (i, 0)),
            pl.BlockSpec((_BN, 2 * _H), lambda i: (i, 0)),
        ],
        out_shape=[
            jax.ShapeDtypeStruct((n, m), jnp.float32),
            jax.ShapeDtypeStruct((n, 2 * _H), jnp.float32),
        ],
    )(x, w, aatt)


def _normalize(raw_ref, asum_ref, b_ref):
    acc = None
    for h in range(_H):
        part = jnp.concatenate([raw_ref[2 * h], raw_ref[2 * h + 1]], axis=1)
        d = asum_ref[...][:, h:h + 1] + 1e-16
        t = part / d
        acc = t if acc is None else acc + t
    hcur = acc * (1.0 / _H) + b_ref[...]
    return jnp.where(hcur > 0, hcur, jnp.exp(jnp.minimum(hcur, 0.0)) - 1.0)


def _np_body(raw_ref, asum_ref, b_ref, w_ref, aatt_ref, h_ref, a_ref):
    hcur = _normalize(raw_ref, asum_ref, b_ref)
    h = jnp.dot(hcur, w_ref[...], preferred_element_type=jnp.float32)
    h_ref[...] = h
    a_ref[...] = jnp.dot(h, aatt_ref[...], preferred_element_type=jnp.float32)


def _norm_project(raw, asum, b, w, aatt):
    m = w.shape[1]
    return pl.pallas_call(
        _np_body,
        grid=(_N // _BN,),
        in_specs=[
            pl.BlockSpec((8, _BN, 128), lambda i: (0, i, 0)),
            pl.BlockSpec((_BN, _H), lambda i: (i, 0)),
            pl.BlockSpec((1, _C), lambda i: (0, 0)),
            pl.BlockSpec((_C, m), lambda i: (0, 0)),
            pl.BlockSpec((m, 2 * _H), lambda i: (0, 0)),
        ],
        out_specs=[
            pl.BlockSpec((_BN, m), lambda i: (i, 0)),
            pl.BlockSpec((_BN, 2 * _H), lambda i: (i, 0)),
        ],
        out_shape=[
            jax.ShapeDtypeStruct((_N, m), jnp.float32),
            jax.ShapeDtypeStruct((_N, 2 * _H), jnp.float32),
        ],
    )(raw, asum, b, w, aatt)


def _final_body(raw_ref, asum_ref, b_ref, fcw_ref, fcb_ref, out_ref, acc_ref):
    i = pl.program_id(0)
    hcur = _normalize(raw_ref, asum_ref, b_ref)
    ssum = jnp.sum(hcur, axis=0, keepdims=True)

    @pl.when(i == 0)
    def _():
        acc_ref[...] = ssum

    @pl.when(i > 0)
    def _():
        acc_ref[...] = acc_ref[...] + ssum

    @pl.when(i == (_N // _BN) - 1)
    def _():
        g = acc_ref[...] * (1.0 / _N)
        out_ref[...] = (jnp.dot(g, fcw_ref[...],
                                preferred_element_type=jnp.float32)
                        + fcb_ref[...])


def _final(raw, asum, b, fc_w, fc_b):
    return pl.pallas_call(
        _final_body,
        grid=(_N // _BN,),
        in_specs=[
            pl.BlockSpec((8, _BN, 128), lambda i: (0, i, 0)),
            pl.BlockSpec((_BN, _H), lambda i: (i, 0)),
            pl.BlockSpec((1, _C), lambda i: (0, 0)),
            pl.BlockSpec((_C, _OUT), lambda i: (0, 0)),
            pl.BlockSpec((1, _OUT), lambda i: (0, 0)),
        ],
        out_specs=pl.BlockSpec((1, _OUT), lambda i: (0, 0)),
        out_shape=jax.ShapeDtypeStruct((1, _OUT), jnp.float32),
        scratch_shapes=[pltpu.VMEM((1, _C), jnp.float32)],
    )(raw, asum, b, fc_w, fc_b)


def kernel(x, edge_index, W1, att_src1, att_dst1, b1, W2, att_src2, att_dst2,
           b2, fc_w, fc_b):
    loop = jnp.arange(_N, dtype=edge_index.dtype)
    src = jnp.concatenate([edge_index[0], loop])
    dst = jnp.concatenate([edge_index[1], loop])
    src_p = jnp.pad(src.astype(jnp.int32), (0, _EPAD - _EREAL))
    dst_p = jnp.pad(dst.astype(jnp.int32), (0, _EPAD - _EREAL))
    src_ra = src_p.reshape(32, _NCA, 128)
    dst_ra = dst_p.reshape(32, _NCA, 128)
    src_rb = src_p.reshape(16, _NSEC, _SCH, 128)
    dst_rb = dst_p.reshape(16, _NSEC, _SCH, 128)

    aatt1 = _build_aatt(att_src1, att_dst1)
    aatt2 = _build_aatt(att_src2, att_dst2)

    def layer(h_pre, a):
        w2, asum_p = _edge_weights(a, src_ra, dst_ra)
        asum = asum_p[0, :_N, :_H] + asum_p[1, :_N, :_H]
        raw = _message_pass(h_pre.reshape(_N * 2 * _H, 128),
                            src_rb, dst_rb, w2)
        return raw[:, :_N, :], asum

    h1p, a1 = _project(x, W1, aatt1)
    raw1, asum1 = layer(h1p, a1)
    h2p, a2 = _norm_project(raw1, asum1, b1.reshape(1, _C), W2, aatt2)
    raw2, asum2 = layer(h2p, a2)
    return _final(raw2, asum2, b2.reshape(1, _C), fc_w, fc_b.reshape(1, _OUT))
